# Initial kernel scaffold; baseline (speedup 1.0000x reference)
import jax
import jax.numpy as jnp
from jax.experimental import pallas as pl


def kernel(x, w1, b1, w2, b2, w3, b3, w4, b4, w5, b5, bn1_scale, bn1_shift, bn2_scale, bn2_shift, bn3_scale, bn3_shift, bn4_scale, bn4_shift):
    raise NotImplementedError("write your pallas kernel here")


# speedup vs baseline: 1.5330x; 1.5330x over previous
import functools

import jax
import jax.numpy as jnp
from jax.experimental import pallas as pl
from jax.experimental.pallas import tpu as pltpu

_NEG = 0.01

_OFFS_A = ((0, 0), (0, 1), (0, 2), (1, 0), (1, 1), (1, 2),
           (2, 0), (2, 1), (2, 2), (1, 1))
_OFFS_B = ((0, 0), (0, 2), (0, 4), (2, 0), (2, 2), (2, 4),
           (4, 0), (4, 2), (4, 4))
_OFFS_C = ((0, 0), (0, 2), (2, 0), (2, 2))


def _lrelu(v):
    return jnp.where(v > 0, v, _NEG * v)


def _body(x_hbm, wa, wb, wb4, wc, wd12, wd3,
          b2, b1, b3, b4, b5, s1, t1, s2, t2, s3, t3, s4, t4,
          oa, ob, xbuf, a1, a2, pbuf, sem, *, th, w, nr):
    n = pl.program_id(0)
    r = pl.program_id(1)
    b2, b1, b3, b4, b5 = b2[...], b1[...], b3[...], b4[...], b5[...]
    s1, t1, s2, t2 = s1[...], t1[...], s2[...], t2[...]
    s3, t3, s4, t4 = s3[...], t3[...], s4[...], t4[...]

    cp = pltpu.make_async_copy(x_hbm.at[n, pl.ds(r * th, th + 10)], xbuf,
                               sem.at[0])
    cp.start()
    cp.wait()

    ra = th + 8
    ma = ra * w
    acc = jnp.zeros((ma, 256), jnp.float32)
    for i in range(5):
        (i1, j1), (i2, j2) = _OFFS_A[2 * i], _OFFS_A[2 * i + 1]
        p = jnp.concatenate(
            [xbuf[i1:i1 + ra, j1:j1 + w, :], xbuf[i2:i2 + ra, j2:j2 + w, :]],
            axis=-1).reshape(ma, 256)
        acc = acc + jnp.dot(p, wa[i], preferred_element_type=jnp.float32)
    y = acc.reshape(ra, w, 256)
    a1v = _lrelu(y[:, :, :128] + b2) * s1 + t1
    a1[:, 2:w + 2, :] = a1v.astype(jnp.bfloat16)
    a1[:, 0:2, :] = jnp.zeros((ra, 2, 128), jnp.bfloat16)
    a1[:, w + 2:w + 4, :] = jnp.zeros((ra, 2, 128), jnp.bfloat16)

    @pl.when(r == 0)
    def _():
        a1[0:4, :, :] = jnp.zeros((4, w + 4, 128), jnp.bfloat16)

    @pl.when(r == nr - 1)
    def _():
        a1[th + 4:th + 8, :, :] = jnp.zeros((4, w + 4, 128), jnp.bfloat16)

    sc = _lrelu(y[3:th + 5, :, 128:] + b1).astype(jnp.bfloat16)

    rb = th + 4
    mb = rb * w
    accb = jnp.zeros((mb, 128), jnp.float32)
    for i in range(4):
        (i1, j1), (i2, j2) = _OFFS_B[2 * i], _OFFS_B[2 * i + 1]
        p = jnp.concatenate(
            [a1[i1:i1 + rb, j1:j1 + w, :], a1[i2:i2 + rb, j2:j2 + w, :]],
            axis=-1).reshape(mb, 256)
        accb = accb + jnp.dot(p, wb[i], preferred_element_type=jnp.float32)
    accb = accb + jnp.dot(a1[4:4 + rb, 4:4 + w, :].reshape(mb, 128), wb4[...],
                          preferred_element_type=jnp.float32)
    a2v = _lrelu(accb.reshape(rb, w, 128) + b3) * s2 + t2
    a2[:, 1:w + 1, :] = a2v.astype(jnp.bfloat16)
    a2[:, 0:1, :] = jnp.zeros((rb, 1, 128), jnp.bfloat16)
    a2[:, w + 1:w + 2, :] = jnp.zeros((rb, 1, 128), jnp.bfloat16)

    @pl.when(r == 0)
    def _():
        a2[0:2, :, :] = jnp.zeros((2, w + 2, 128), jnp.bfloat16)

    @pl.when(r == nr - 1)
    def _():
        a2[th + 2:th + 4, :, :] = jnp.zeros((2, w + 2, 128), jnp.bfloat16)

    rc = th + 2
    mc = rc * w
    pc0 = jnp.concatenate(
        [a2[0:rc, 0:w, :], a2[0:rc, 2:2 + w, :]], axis=-1).reshape(mc, 256)
    pc1 = jnp.concatenate(
        [a2[2:2 + rc, 0:w, :], a2[2:2 + rc, 2:2 + w, :]],
        axis=-1).reshape(mc, 256)
    accc = (jnp.dot(pc0, wc[0], preferred_element_type=jnp.float32)
            + jnp.dot(pc1, wc[1], preferred_element_type=jnp.float32))
    a3 = (_lrelu(accc.reshape(rc, w, 128) + b4) * s3 + t3).astype(jnp.bfloat16)

    p12 = jnp.concatenate(
        [a1[3:3 + rc, 2:2 + w, :], a2[1:1 + rc, 1:1 + w, :]],
        axis=-1).reshape(mc, 256)
    accd = (jnp.dot(p12, wd12[...], preferred_element_type=jnp.float32)
            + jnp.dot(a3.reshape(mc, 128), wd3[...],
                      preferred_element_type=jnp.float32))
    resa = _lrelu(accd + b5) * s4 + t4 + sc.reshape(mc, 128).astype(jnp.float32)
    resa = resa.reshape(rc, w, 128)
    oa[...] = resa[1:1 + th, :, :oa.shape[-1]]

    pbuf[:, 1:1 + w // 2, :] = resa.reshape(rc, w // 2, 256)
    pbuf[:, 0:1, :] = jnp.zeros((rc, 1, 256), jnp.float32)

    @pl.when(r == 0)
    def _():
        pbuf[0:1, :, :] = jnp.zeros((1, w // 2 + 1, 256), jnp.float32)

    pv = pbuf[...]
    ev = pv[:, 1:1 + w // 2, 0:128]
    od = pv[:, 1:1 + w // 2, 128:256]
    osh = pv[:, 0:w // 2, 128:256]
    hsum = (ev + od + osh).reshape(rc // 2, 2, w // 2, 128)
    vsum = (hsum[0:th // 2, 0] + hsum[0:th // 2, 1]
            + hsum[1:1 + th // 2, 0])
    ob[...] = (vsum * (1.0 / 9.0))[:, :, :ob.shape[-1]]


def kernel(x, w1, b1, w2, b2, w3, b3, w4, b4, w5, b5,
           bn1_scale, bn1_shift, bn2_scale, bn2_shift,
           bn3_scale, bn3_shift, bn4_scale, bn4_shift):
    n, cin, h, w = x.shape
    cout = w1.shape[-1]
    th = max(d for d in range(2, min(h, 16) + 1, 2) if h % d == 0)
    nr = h // th
    bf16 = jnp.bfloat16

    w2r = jnp.pad(w2.reshape(9, cin, cout),
                  ((0, 0), (0, 128 - cin), (0, 128 - cout))).astype(bf16)
    w1r = jnp.pad(w1.reshape(cin, cout),
                  ((0, 128 - cin), (0, 128 - cout))).astype(bf16)
    zero = jnp.zeros((128, 128), bf16)

    def ablock(t):
        left = w2r[t] if t < 9 else zero
        right = w1r if t == 9 else zero
        return jnp.concatenate([left, right], axis=1)

    wa = jnp.stack([jnp.concatenate([ablock(2 * i), ablock(2 * i + 1)], axis=0)
                    for i in range(5)])

    w3r = jnp.pad(w3.reshape(9, cout, cout),
                  ((0, 0), (0, 128 - cout), (0, 128 - cout))).astype(bf16)
    wb = jnp.stack([jnp.concatenate([w3r[2 * i], w3r[2 * i + 1]], axis=0)
                    for i in range(4)])
    wb4 = w3r[8]

    w4r = jnp.pad(w4.reshape(4, cout, cout),
                  ((0, 0), (0, 128 - cout), (0, 128 - cout))).astype(bf16)
    wc = jnp.stack([jnp.concatenate([w4r[0], w4r[1]], axis=0),
                    jnp.concatenate([w4r[2], w4r[3]], axis=0)])

    w5r = jnp.pad(w5.reshape(3, cout, cout),
                  ((0, 0), (0, 128 - cout), (0, 128 - cout))).astype(bf16)
    wd12 = jnp.concatenate([w5r[0], w5r[1]], axis=0)
    wd3 = w5r[2]

    def vec(v, fill=0.0):
        return jnp.pad(v, ((0, 0), (0, 128 - cout)),
                       constant_values=fill).astype(jnp.float32)

    b1p, b2p, b3p, b4p, b5p = vec(b1), vec(b2), vec(b3), vec(b4), vec(b5)
    s1, t1 = vec(bn1_scale, 1.0), vec(bn1_shift)
    s2, t2 = vec(bn2_scale, 1.0), vec(bn2_shift)
    s3, t3 = vec(bn3_scale, 1.0), vec(bn3_shift)
    s4, t4 = vec(bn4_scale, 1.0), vec(bn4_shift)

    xp = jnp.pad(jnp.transpose(x, (0, 2, 3, 1)),
                 ((0, 0), (5, 5), (1, 1), (0, 128 - cin))).astype(bf16)

    def wspec(shape):
        return pl.BlockSpec(shape, lambda i, j: (0,) * len(shape))

    vspec = pl.BlockSpec((1, 128), lambda i, j: (0, 0))
    body = functools.partial(_body, th=th, w=w, nr=nr)
    ra_, rb_ = pl.pallas_call(
        body,
        out_shape=(jax.ShapeDtypeStruct((n, h, w, cout), jnp.float32),
                   jax.ShapeDtypeStruct((n, h // 2, w // 2, cout),
                                        jnp.float32)),
        grid=(n, nr),
        in_specs=[pl.BlockSpec(memory_space=pl.ANY),
                  wspec((5, 256, 256)), wspec((4, 256, 128)),
                  wspec((128, 128)), wspec((2, 256, 128)),
                  wspec((256, 128)), wspec((128, 128)),
                  vspec, vspec, vspec, vspec, vspec,
                  vspec, vspec, vspec, vspec, vspec, vspec, vspec, vspec],
        out_specs=(pl.BlockSpec((None, th, w, cout), lambda i, j: (i, j, 0, 0)),
                   pl.BlockSpec((None, th // 2, w // 2, cout),
                                lambda i, j: (i, j, 0, 0))),
        scratch_shapes=[pltpu.VMEM((th + 10, w + 2, 128), bf16),
                        pltpu.VMEM((th + 8, w + 4, 128), bf16),
                        pltpu.VMEM((th + 4, w + 2, 128), bf16),
                        pltpu.VMEM((th + 2, w // 2 + 1, 256), jnp.float32),
                        pltpu.SemaphoreType.DMA((1,))],
        compiler_params=pltpu.CompilerParams(
            dimension_semantics=("parallel", "parallel"),
            vmem_limit_bytes=64 * 1024 * 1024),
    )(xp, wa, wb, wb4, wc, wd12, wd3,
      b2p, b1p, b3p, b4p, b5p, s1, t1, s2, t2, s3, t3, s4, t4)

    return (jnp.transpose(rb_, (0, 3, 1, 2)),
            jnp.transpose(ra_, (0, 3, 1, 2)))


# speedup vs baseline: 2.7765x; 1.8112x over previous
import functools

import jax
import jax.numpy as jnp
from jax.experimental import pallas as pl
from jax.experimental.pallas import tpu as pltpu

_NEG = 0.01


def _lrelu(v):
    return jnp.where(v > 0, v, _NEG * v)


def _body(x_hbm, waa, wab, wb, wb4, wc, wd12, wd3,
          b2, b1, b3, b4, b5, s1, t1, s2, t2, s3, t3, s4, t4,
          oa, ob, xbuf, pbuf, sem, *, th, w, h, nr):
    n = pl.program_id(0)
    r = pl.program_id(1)
    b2, b1, b3, b4, b5 = b2[...], b1[...], b3[...], b4[...], b5[...]
    s1, t1, s2, t2 = s1[...], t1[...], s2[...], t2[...]
    s3, t3, s4, t4 = s3[...], t3[...], s4[...], t4[...]
    bf16 = jnp.bfloat16

    cp = pltpu.make_async_copy(x_hbm.at[n, pl.ds(r * th, th + 10)], xbuf,
                               sem.at[0])
    cp.start()
    cp.wait()

    ra = th + 8
    ma = ra * w
    pa = jnp.concatenate([xbuf[0:ra], xbuf[1:1 + ra]], axis=-1).reshape(ma, 256)
    acc = (jnp.dot(pa, waa[...], preferred_element_type=jnp.float32)
           + jnp.dot(xbuf[2:2 + ra].reshape(ma, 128), wab[...],
                     preferred_element_type=jnp.float32))
    y = acc.reshape(ra, w, 256)
    sc = _lrelu(y[3:th + 5, :, 128:] + b1).astype(bf16)
    a1v = _lrelu(y[:, :, :128] + b2) * s1 + t1
    gi = jax.lax.broadcasted_iota(jnp.int32, (ra, 1, 1), 0) + (r * th - 4)
    a1b = jnp.where((gi >= 0) & (gi < h), a1v, 0.0).astype(bf16)

    z2 = jnp.zeros((ra, 2, 128), bf16)
    a1m2 = jnp.concatenate([z2, a1b[:, :w - 2, :]], axis=1)
    a1p2 = jnp.concatenate([a1b[:, 2:, :], z2], axis=1)
    rb = th + 4
    mb = rb * w

    def cat2(u, v):
        return jnp.concatenate([u, v], axis=-1).reshape(u.shape[0] * w, 256)

    accb = (jnp.dot(cat2(a1m2[0:rb], a1b[0:rb]), wb[0],
                    preferred_element_type=jnp.float32)
            + jnp.dot(cat2(a1p2[0:rb], a1m2[2:2 + rb]), wb[1],
                      preferred_element_type=jnp.float32)
            + jnp.dot(cat2(a1b[2:2 + rb], a1p2[2:2 + rb]), wb[2],
                      preferred_element_type=jnp.float32)
            + jnp.dot(cat2(a1m2[4:4 + rb], a1b[4:4 + rb]), wb[3],
                      preferred_element_type=jnp.float32)
            + jnp.dot(a1p2[4:4 + rb].reshape(mb, 128), wb4[...],
                      preferred_element_type=jnp.float32))
    a2v = _lrelu(accb.reshape(rb, w, 128) + b3) * s2 + t2
    gj = jax.lax.broadcasted_iota(jnp.int32, (rb, 1, 1), 0) + (r * th - 2)
    a2b = jnp.where((gj >= 0) & (gj < h), a2v, 0.0).astype(bf16)

    z1 = jnp.zeros((rb, 1, 128), bf16)
    a2m1 = jnp.concatenate([z1, a2b[:, :w - 1, :]], axis=1)
    a2p1 = jnp.concatenate([a2b[:, 1:, :], z1], axis=1)
    rc = th + 2
    mc = rc * w
    accc = (jnp.dot(cat2(a2m1[0:rc], a2p1[0:rc]), wc[0],
                    preferred_element_type=jnp.float32)
            + jnp.dot(cat2(a2m1[2:2 + rc], a2p1[2:2 + rc]), wc[1],
                      preferred_element_type=jnp.float32))
    a3 = (_lrelu(accc.reshape(rc, w, 128) + b4) * s3 + t3).astype(bf16)

    p12 = jnp.concatenate([a1b[3:3 + rc], a2b[1:1 + rc]],
                          axis=-1).reshape(mc, 256)
    accd = (jnp.dot(p12, wd12[...], preferred_element_type=jnp.float32)
            + jnp.dot(a3.reshape(mc, 128), wd3[...],
                      preferred_element_type=jnp.float32))
    resa = _lrelu(accd + b5) * s4 + t4 + sc.reshape(mc, 128).astype(jnp.float32)
    resa = resa.reshape(rc, w, 128)
    oa[...] = resa[1:1 + th, :, :oa.shape[-1]]

    pbuf[:, 1:1 + w // 2, :] = resa.reshape(rc, w // 2, 256)
    pbuf[:, 0:1, :] = jnp.zeros((rc, 1, 256), jnp.float32)

    @pl.when(r == 0)
    def _():
        pbuf[0:1, :, :] = jnp.zeros((1, w // 2 + 1, 256), jnp.float32)

    pv = pbuf[...]
    ev = pv[:, 1:1 + w // 2, 0:128]
    od = pv[:, 1:1 + w // 2, 128:256]
    osh = pv[:, 0:w // 2, 128:256]
    hsum = (ev + od + osh).reshape(rc // 2, 2, w // 2, 128)
    vsum = (hsum[0:th // 2, 0] + hsum[0:th // 2, 1]
            + hsum[1:1 + th // 2, 0])
    ob[...] = (vsum * (1.0 / 9.0))[:, :, :ob.shape[-1]]


def kernel(x, w1, b1, w2, b2, w3, b3, w4, b4, w5, b5,
           bn1_scale, bn1_shift, bn2_scale, bn2_shift,
           bn3_scale, bn3_shift, bn4_scale, bn4_shift):
    n, cin, h, w = x.shape
    cout = w1.shape[-1]
    th = max(d for d in range(2, min(h, 16) + 1, 2) if h % d == 0)
    nr = h // th
    bf16 = jnp.bfloat16

    def padc(m):
        return jnp.pad(m, ((0, 0), (0, 128 - cout)))

    zk = jnp.zeros((128 - 3 * cin, cout), jnp.float32)

    def ablock(di):
        left = jnp.concatenate([w2[di, 0], w2[di, 1], w2[di, 2], zk], axis=0)
        if di == 1:
            right = jnp.concatenate(
                [jnp.zeros((cin, cout), jnp.float32), w1[0, 0],
                 jnp.zeros((128 - 2 * cin, cout), jnp.float32)], axis=0)
        else:
            right = jnp.zeros((128, cout), jnp.float32)
        return jnp.concatenate([padc(left), padc(right)], axis=1)

    waa = jnp.concatenate([ablock(0), ablock(1)], axis=0).astype(bf16)
    wab = ablock(2).astype(bf16)

    w3r = jnp.pad(w3.reshape(9, cout, cout),
                  ((0, 0), (0, 128 - cout), (0, 128 - cout))).astype(bf16)
    wb = jnp.stack([jnp.concatenate([w3r[2 * i], w3r[2 * i + 1]], axis=0)
                    for i in range(4)])
    wb4 = w3r[8]

    w4r = jnp.pad(w4.reshape(4, cout, cout),
                  ((0, 0), (0, 128 - cout), (0, 128 - cout))).astype(bf16)
    wc = jnp.stack([jnp.concatenate([w4r[0], w4r[1]], axis=0),
                    jnp.concatenate([w4r[2], w4r[3]], axis=0)])

    w5r = jnp.pad(w5.reshape(3, cout, cout),
                  ((0, 0), (0, 128 - cout), (0, 128 - cout))).astype(bf16)
    wd12 = jnp.concatenate([w5r[0], w5r[1]], axis=0)
    wd3 = w5r[2]

    def vec(v, fill=0.0):
        return jnp.pad(v, ((0, 0), (0, 128 - cout)),
                       constant_values=fill).astype(jnp.float32)

    b1p, b2p, b3p, b4p, b5p = vec(b1), vec(b2), vec(b3), vec(b4), vec(b5)
    s1, t1 = vec(bn1_scale, 1.0), vec(bn1_shift)
    s2, t2 = vec(bn2_scale, 1.0), vec(bn2_shift)
    s3, t3 = vec(bn3_scale, 1.0), vec(bn3_shift)
    s4, t4 = vec(bn4_scale, 1.0), vec(bn4_shift)

    xn = jnp.transpose(x, (0, 2, 3, 1))
    xl = jnp.pad(xn, ((0, 0), (0, 0), (1, 0), (0, 0)))[:, :, :w, :]
    xr = jnp.pad(xn, ((0, 0), (0, 0), (0, 1), (0, 0)))[:, :, 1:, :]
    x3 = jnp.concatenate(
        [xl, xn, xr, jnp.zeros(xn.shape[:3] + (128 - 3 * cin,), xn.dtype)],
        axis=-1)
    xp = jnp.pad(x3, ((0, 0), (5, 5), (0, 0), (0, 0))).astype(bf16)

    def wspec(shape):
        return pl.BlockSpec(shape, lambda i, j: (0,) * len(shape))

    vspec = pl.BlockSpec((1, 128), lambda i, j: (0, 0))
    body = functools.partial(_body, th=th, w=w, h=h, nr=nr)
    ra_, rb_ = pl.pallas_call(
        body,
        out_shape=(jax.ShapeDtypeStruct((n, h, w, cout), jnp.float32),
                   jax.ShapeDtypeStruct((n, h // 2, w // 2, cout),
                                        jnp.float32)),
        grid=(n, nr),
        in_specs=[pl.BlockSpec(memory_space=pl.ANY),
                  wspec((256, 256)), wspec((128, 256)),
                  wspec((4, 256, 128)), wspec((128, 128)),
                  wspec((2, 256, 128)), wspec((256, 128)), wspec((128, 128)),
                  vspec, vspec, vspec, vspec, vspec,
                  vspec, vspec, vspec, vspec, vspec, vspec, vspec, vspec],
        out_specs=(pl.BlockSpec((None, th, w, cout), lambda i, j: (i, j, 0, 0)),
                   pl.BlockSpec((None, th // 2, w // 2, cout),
                                lambda i, j: (i, j, 0, 0))),
        scratch_shapes=[pltpu.VMEM((th + 10, w, 128), bf16),
                        pltpu.VMEM((th + 2, w // 2 + 1, 256), jnp.float32),
                        pltpu.SemaphoreType.DMA((1,))],
        compiler_params=pltpu.CompilerParams(
            dimension_semantics=("parallel", "parallel"),
            vmem_limit_bytes=64 * 1024 * 1024),
    )(xp, waa, wab, wb, wb4, wc, wd12, wd3,
      b2p, b1p, b3p, b4p, b5p, s1, t1, s2, t2, s3, t3, s4, t4)

    return (jnp.transpose(rb_, (0, 3, 1, 2)),
            jnp.transpose(ra_, (0, 3, 1, 2)))
